# Initial kernel scaffold; baseline (speedup 1.0000x reference)
import jax
import jax.numpy as jnp
from jax.experimental import pallas as pl


def kernel(x, edge_index, Wc, bc, W1, b1, W2, b2, W3, b3):
    raise NotImplementedError("write your pallas kernel here")


# speedup vs baseline: 13.8818x; 13.8818x over previous
import functools

import jax
import jax.numpy as jnp
from jax import lax
from jax.experimental import pallas as pl
from jax.experimental.pallas import tpu as pltpu
from jax.experimental.pallas import tpu_sc as plsc

N, E, D = 10000, 320000, 128
OUT = 10

NC, NS = 2, 16
NW = NC * NS
CH = 128
NCHUNK = 79
EW = CH * NCHUNK
EPAD = EW * NW
NPAD = 10240
RPT = NPAD // NS
NTRASH = NPAD - N

_mesh = plsc.VectorSubcoreMesh(core_axis_name="c", subcore_axis_name="s")


@functools.partial(
    pl.kernel,
    mesh=_mesh,
    out_type=[
        jax.ShapeDtypeStruct((NC, NPAD), jnp.float32),
        jax.ShapeDtypeStruct((NC, NPAD), jnp.float32),
    ],
    scratch_types=[
        pltpu.VMEM((CH,), jnp.int32),
        pltpu.VMEM((CH,), jnp.float32),
        pltpu.VMEM((RPT,), jnp.float32),
        pltpu.VMEM_SHARED((NPAD,), jnp.float32),
        pltpu.VMEM_SHARED((NPAD,), jnp.float32),
    ],
)
def _deg_kernel(src_hbm, dst_hbm, dout_hbm, din_hbm, idx_v, ones_v, stage_v,
                do_sh, di_sh):
    c = lax.axis_index("c")
    s = lax.axis_index("s")
    wid = s * NC + c
    for i in range(CH // 16):
        ones_v[pl.ds(i * 16, 16)] = jnp.ones((16,), jnp.float32)
    for i in range(RPT // 16):
        stage_v[pl.ds(i * 16, 16)] = jnp.zeros((16,), jnp.float32)
    pltpu.sync_copy(stage_v, do_sh.at[pl.ds(s * RPT, RPT)])
    pltpu.sync_copy(stage_v, di_sh.at[pl.ds(s * RPT, RPT)])
    plsc.subcore_barrier()
    base = wid * EW

    def body(j, carry):
        off = base + j * CH
        pltpu.sync_copy(src_hbm.at[pl.ds(off, CH)], idx_v)
        pltpu.sync_copy(ones_v, do_sh.at[idx_v], add=True)
        pltpu.sync_copy(dst_hbm.at[pl.ds(off, CH)], idx_v)
        pltpu.sync_copy(ones_v, di_sh.at[idx_v], add=True)
        return carry

    lax.fori_loop(0, NCHUNK, body, 0)
    plsc.subcore_barrier()
    sl = pl.ds(s * RPT, RPT)
    pltpu.sync_copy(do_sh.at[sl], stage_v)
    pltpu.sync_copy(stage_v, dout_hbm.at[c, sl])
    pltpu.sync_copy(di_sh.at[sl], stage_v)
    pltpu.sync_copy(stage_v, din_hbm.at[c, sl])


@functools.partial(
    pl.kernel,
    mesh=_mesh,
    out_type=jax.ShapeDtypeStruct((NC, NPAD, D), jnp.float32),
    scratch_types=[
        pltpu.VMEM((CH,), jnp.int32),
        pltpu.VMEM((CH,), jnp.int32),
        pltpu.VMEM((CH, D), jnp.float32),
        pltpu.SemaphoreType.DMA,
        pltpu.VMEM_SHARED((NPAD, D), jnp.float32),
    ],
)
def _scatter_kernel(h_hbm, src_hbm, dst_hbm, out_hbm, si_v, di_v, rows_v, sem,
                    agg_sh):
    c = lax.axis_index("c")
    s = lax.axis_index("s")
    wid = s * NC + c

    def zrow(r, carry):
        for k in range(D // 16):
            rows_v[r, pl.ds(k * 16, 16)] = jnp.zeros((16,), jnp.float32)
        return carry

    lax.fori_loop(0, CH, zrow, 0)
    for k in range(RPT // CH):
        pltpu.sync_copy(rows_v, agg_sh.at[pl.ds(s * RPT + k * CH, CH)])
    plsc.subcore_barrier()

    base = wid * EW

    def body(j, carry):
        off = base + j * CH
        pltpu.sync_copy(src_hbm.at[pl.ds(off, CH)], si_v)
        pltpu.sync_copy(dst_hbm.at[pl.ds(off, CH)], di_v)
        pltpu.async_copy(h_hbm.at[si_v], rows_v, sem).wait()
        pltpu.sync_copy(rows_v, agg_sh.at[di_v], add=True)
        return carry

    lax.fori_loop(0, NCHUNK, body, 0)
    plsc.subcore_barrier()
    for k in range(RPT // CH):
        sl = pl.ds(s * RPT + k * CH, CH)
        pltpu.sync_copy(agg_sh.at[sl], rows_v)
        pltpu.sync_copy(rows_v, out_hbm.at[c, sl])


BLK2 = 512


def _mm_body(x_ref, wc_ref, dout_ref, din_ref, h_ref, nd_ref):
    do = dout_ref[0] + dout_ref[1] + 1.0
    di = din_ref[0] + din_ref[1] + 1.0
    ns = lax.rsqrt(do)
    nd_ref[...] = lax.rsqrt(di)
    h_ref[...] = jnp.dot(x_ref[...], wc_ref[...],
                         preferred_element_type=jnp.float32) * ns


BLK4 = 1024
G4 = NPAD // BLK4


def _head_body(part_ref, h_ref, nd_ref, bc_ref, w1_ref, b1_ref, w2_ref,
               b2_ref, w3_ref, b3_ref, out_ref, mx_ref):
    i = pl.program_id(0)
    v = (part_ref[0] + part_ref[1] + h_ref[...]) * nd_ref[...] + bc_ref[...]
    v = jnp.maximum(v, 0.0)
    rows = lax.broadcasted_iota(jnp.int32, (BLK4, 1), 0) + i * BLK4
    v = jnp.where(rows < N, v, 0.0)
    bmax = jnp.max(v, axis=0, keepdims=True)

    @pl.when(i == 0)
    def _():
        mx_ref[...] = bmax

    @pl.when(i > 0)
    def _():
        mx_ref[...] = jnp.maximum(mx_ref[...], bmax)

    @pl.when(i == G4 - 1)
    def _():
        hg = mx_ref[...]
        a = jnp.maximum(
            jnp.dot(hg, w1_ref[...], preferred_element_type=jnp.float32)
            + b1_ref[...], 0.0)
        a = jnp.maximum(
            jnp.dot(a, w2_ref[...], preferred_element_type=jnp.float32)
            + b2_ref[...], 0.0)
        out_ref[...] = jnp.dot(a, w3_ref[...],
                               preferred_element_type=jnp.float32) + b3_ref[...]


def kernel(x, edge_index, Wc, bc, W1, b1, W2, b2, W3, b3):
    src = edge_index[0]
    dst = edge_index[1]
    pad = (jnp.arange(EPAD - E, dtype=jnp.int32) % NTRASH) + N
    src_p = jnp.concatenate([src, pad])
    dst_p = jnp.concatenate([dst, pad])
    x_pad = jnp.concatenate(
        [x, jnp.zeros((NPAD - N, D), jnp.float32)], axis=0)

    dout, din = _deg_kernel(src_p, dst_p)

    h_scaled, norm_dst = pl.pallas_call(
        _mm_body,
        grid=(NPAD // BLK2,),
        in_specs=[
            pl.BlockSpec((BLK2, D), lambda i: (i, 0)),
            pl.BlockSpec((D, D), lambda i: (0, 0)),
            pl.BlockSpec((NC, BLK2, 1), lambda i: (0, i, 0)),
            pl.BlockSpec((NC, BLK2, 1), lambda i: (0, i, 0)),
        ],
        out_specs=[
            pl.BlockSpec((BLK2, D), lambda i: (i, 0)),
            pl.BlockSpec((BLK2, 1), lambda i: (i, 0)),
        ],
        out_shape=[
            jax.ShapeDtypeStruct((NPAD, D), jnp.float32),
            jax.ShapeDtypeStruct((NPAD, 1), jnp.float32),
        ],
    )(x_pad, Wc, dout.reshape(NC, NPAD, 1), din.reshape(NC, NPAD, 1))

    part = _scatter_kernel(h_scaled, src_p, dst_p)

    out = pl.pallas_call(
        _head_body,
        grid=(G4,),
        in_specs=[
            pl.BlockSpec((NC, BLK4, D), lambda i: (0, i, 0)),
            pl.BlockSpec((BLK4, D), lambda i: (i, 0)),
            pl.BlockSpec((BLK4, 1), lambda i: (i, 0)),
            pl.BlockSpec((1, D), lambda i: (0, 0)),
            pl.BlockSpec((D, 256), lambda i: (0, 0)),
            pl.BlockSpec((1, 256), lambda i: (0, 0)),
            pl.BlockSpec((256, D), lambda i: (0, 0)),
            pl.BlockSpec((1, D), lambda i: (0, 0)),
            pl.BlockSpec((D, OUT), lambda i: (0, 0)),
            pl.BlockSpec((1, OUT), lambda i: (0, 0)),
        ],
        out_specs=pl.BlockSpec((1, OUT), lambda i: (0, 0)),
        out_shape=jax.ShapeDtypeStruct((1, OUT), jnp.float32),
        scratch_shapes=[pltpu.VMEM((1, D), jnp.float32)],
    )(part, h_scaled, norm_dst, bc.reshape(1, D), W1, b1.reshape(1, 256),
      W2, b2.reshape(1, D), W3, b3.reshape(1, OUT))

    return jnp.squeeze(out)
